# Initial kernel scaffold; baseline (speedup 1.0000x reference)
import jax
import jax.numpy as jnp
from jax.experimental import pallas as pl


def kernel(inputs, weights, locations):
    raise NotImplementedError("write your pallas kernel here")


# speedup vs baseline: 9.8587x; 9.8587x over previous
import math

import jax
import jax.numpy as jnp
from jax.experimental import pallas as pl

HEIGHT = 32
WIDTH = 32
INPUT_SIZE = 64
NUM_ITERS = 5
LEARNING_RATE = 0.1
BATCH = 1024
RADIUS = max(HEIGHT / 2.0, WIDTH / 2.0)
TIME_CONSTANT = NUM_ITERS / math.log(RADIUS)
K = HEIGHT * WIDTH

_HIGH = jax.lax.Precision.HIGHEST


def _som_body(x_ref, xt_ref, w_ref, out_ref):
    x = x_ref[:]
    xt = xt_ref[:]
    w = w_ref[:]

    krow = jax.lax.broadcasted_iota(jnp.int32, (K, 1), 0)
    ki = (krow >> 5).astype(jnp.float32)
    kj = (krow & 31).astype(jnp.float32)

    bmu = None
    for i in range(NUM_ITERS):
        lr = LEARNING_RATE * math.exp(-i / NUM_ITERS)
        nr = RADIUS * math.exp(-i / TIME_CONSTANT)
        nr2 = nr * nr

        wn = jnp.sum(w * w, axis=1, keepdims=True)
        dots = jax.lax.dot_general(w, xt, (((1,), (0,)), ((), ())),
                                   preferred_element_type=jnp.float32,
                                   precision=_HIGH)
        score = wn - 2.0 * dots

        cmin = jnp.min(score, axis=0, keepdims=True)
        bmu = jnp.min(jnp.where(score == cmin, krow, K), axis=0,
                      keepdims=True).astype(jnp.int32)

        bi = (bmu >> 5).astype(jnp.float32)
        bj = (bmu & 31).astype(jnp.float32)
        di = ki - bi
        dj = kj - bj
        d2 = di * di + dj * dj

        eff = jnp.where(d2 < nr2,
                        lr * jnp.exp(d2 * (-0.5 / nr2)),
                        0.0).astype(jnp.float32)

        s = jnp.sum(eff, axis=1, keepdims=True)
        u = jax.lax.dot_general(eff, x, (((1,), (0,)), ((), ())),
                                preferred_element_type=jnp.float32,
                                precision=_HIGH)
        w = w * (1.0 - s * (1.0 / BATCH)) + u * (1.0 / BATCH)

    bmu_col = jnp.transpose(bmu, (1, 0))
    kcols = jax.lax.broadcasted_iota(jnp.int32, (1, K), 1)
    onehot = (kcols == bmu_col).astype(jnp.float32)
    out_ref[:] = jax.lax.dot_general(onehot, w, (((1,), (0,)), ((), ())),
                                     preferred_element_type=jnp.float32,
                                     precision=_HIGH)


def kernel(inputs, weights, locations):
    del locations
    xt = jnp.transpose(inputs, (1, 0))
    return pl.pallas_call(
        _som_body,
        out_shape=jax.ShapeDtypeStruct((BATCH, INPUT_SIZE), jnp.float32),
    )(inputs, xt, weights)
